# baseline (speedup 1.0000x reference)
import functools

import jax
import jax.numpy as jnp
from jax import lax
from jax.experimental import pallas as pl
from jax.experimental.pallas import tpu as pltpu
from jax.experimental.pallas import tpu_sc as plsc

NC, NS = 2, 16
NW = NC * NS
L = 16
NBUF = 8


def kernel(x, pe):
    B, S, D = x.shape
    s_per_w = S // NW
    CH = 2
    n_chunks = s_per_w // CH

    @functools.partial(
        pl.kernel,
        out_type=jax.ShapeDtypeStruct((B, S, D), jnp.float32),
        mesh=plsc.VectorSubcoreMesh(
            core_axis_name="c", subcore_axis_name="s",
            num_cores=NC, num_subcores=NS),
        scratch_types=[
            pltpu.VMEM((NBUF, CH, D), jnp.float32),
            pltpu.VMEM((NBUF, B, CH, D), jnp.float32),
        ] + [pltpu.SemaphoreType.DMA] * (2 * NBUF),
    )
    def sc_add(x_hbm, pe_hbm, out_hbm, pe_v, x_v, *sems):
        wid = lax.axis_index("s") * NC + lax.axis_index("c")
        base = wid * s_per_w
        in_sems = sems[:NBUF]
        out_sems = sems[NBUF:]

        def in_copies(c, slot):
            s0 = base + c * CH
            pltpu.async_copy(pe_hbm.at[pl.ds(s0, CH)], pe_v.at[slot],
                             in_sems[slot])
            pltpu.async_copy(x_hbm.at[:, pl.ds(s0, CH), :],
                             x_v.at[slot], in_sems[slot])

        def wait_in(slot):
            pltpu.make_async_copy(pe_hbm.at[pl.ds(base, CH)], pe_v.at[slot],
                                  in_sems[slot]).wait()
            pltpu.make_async_copy(x_hbm.at[:, pl.ds(base, CH), :],
                                  x_v.at[slot], in_sems[slot]).wait()

        def out_copies(c, slot):
            s0 = base + c * CH
            pltpu.async_copy(x_v.at[slot],
                             out_hbm.at[:, pl.ds(s0, CH), :],
                             out_sems[slot])

        def wait_out(slot):
            pltpu.make_async_copy(x_v.at[slot],
                                  out_hbm.at[:, pl.ds(base, CH), :],
                                  out_sems[slot]).wait()

        def compute(slot):
            def pos_body(si, carry):
                for dcol in range(D // L):
                    sl = pl.ds(dcol * L, L)
                    vec = pe_v[slot, si, sl]
                    for b in range(B):
                        plsc.addupdate(x_v.at[slot, b, si, sl], vec)
                return carry
            lax.fori_loop(0, CH, pos_body, 0)

        for k in range(NBUF):
            in_copies(k, k)

        def loop_body(i4, carry):
            for k in range(NBUF):
                c = i4 * NBUF + k
                wait_in(k)
                compute(k)
                out_copies(c, k)
                j = (k + 2) % NBUF

                @pl.when(jnp.logical_and(c >= NBUF - 2, c + 2 < n_chunks))
                def _():
                    wait_out(j)
                    in_copies(c + 2, j)
            return carry

        lax.fori_loop(0, n_chunks // NBUF, loop_body, 0)
        for k in range(NBUF):
            wait_out(k)

    return sc_add(x, pe)


# speedup vs baseline: 1.1655x; 1.1655x over previous
import functools

import jax
import jax.numpy as jnp
from jax import lax
from jax.experimental import pallas as pl
from jax.experimental.pallas import tpu as pltpu
from jax.experimental.pallas import tpu_sc as plsc

NC, NS = 2, 16
NW = NC * NS
L = 16
NBUF = 4


def kernel(x, pe):
    B, S, D = x.shape
    s_per_w = S // NW
    CH = 4
    n_chunks = s_per_w // CH

    @functools.partial(
        pl.kernel,
        out_type=jax.ShapeDtypeStruct((B, S, D), jnp.float32),
        mesh=plsc.VectorSubcoreMesh(
            core_axis_name="c", subcore_axis_name="s",
            num_cores=NC, num_subcores=NS),
        scratch_types=[
            pltpu.VMEM((NBUF, CH, D), jnp.float32),
            pltpu.VMEM((NBUF, B, CH, D), jnp.float32),
            pltpu.SemaphoreType.DMA,
            pltpu.SemaphoreType.DMA,
            pltpu.SemaphoreType.DMA,
            pltpu.SemaphoreType.DMA,
            pltpu.SemaphoreType.DMA,
            pltpu.SemaphoreType.DMA,
            pltpu.SemaphoreType.DMA,
            pltpu.SemaphoreType.DMA,
        ],
    )
    def sc_add(x_hbm, pe_hbm, out_hbm, pe_v, x_v,
               in0, in1, in2, in3, ou0, ou1, ou2, ou3):
        wid = lax.axis_index("s") * NC + lax.axis_index("c")
        base = wid * s_per_w
        in_sems = (in0, in1, in2, in3)
        out_sems = (ou0, ou1, ou2, ou3)

        def in_copies(c, slot):
            s0 = base + c * CH
            pltpu.async_copy(pe_hbm.at[pl.ds(s0, CH)], pe_v.at[slot],
                             in_sems[slot])
            pltpu.async_copy(x_hbm.at[:, pl.ds(s0, CH), :],
                             x_v.at[slot], in_sems[slot])

        def wait_in(slot):
            pltpu.make_async_copy(pe_hbm.at[pl.ds(base, CH)], pe_v.at[slot],
                                  in_sems[slot]).wait()
            pltpu.make_async_copy(x_hbm.at[:, pl.ds(base, CH), :],
                                  x_v.at[slot], in_sems[slot]).wait()

        def out_copies(c, slot):
            s0 = base + c * CH
            pltpu.async_copy(x_v.at[slot],
                             out_hbm.at[:, pl.ds(s0, CH), :],
                             out_sems[slot])

        def wait_out(slot):
            pltpu.make_async_copy(x_v.at[slot],
                                  out_hbm.at[:, pl.ds(base, CH), :],
                                  out_sems[slot]).wait()

        def compute(slot):
            def pos_body(si, carry):
                for dcol in range(D // L):
                    sl = pl.ds(dcol * L, L)
                    vec = pe_v[slot, si, sl]
                    for b in range(B):
                        plsc.addupdate(x_v.at[slot, b, si, sl], vec)
                return carry
            lax.fori_loop(0, CH, pos_body, 0)

        for k in range(NBUF):
            in_copies(k, k)

        def loop_body(i4, carry):
            for k in range(NBUF):
                c = i4 * NBUF + k
                wait_in(k)
                j = (k + 2) % NBUF

                @pl.when(jnp.logical_and(c >= 2, c + 2 < n_chunks))
                def _():
                    wait_out(j)
                    in_copies(c + 2, j)

                compute(k)
                out_copies(c, k)
            return carry

        lax.fori_loop(0, n_chunks // NBUF, loop_body, 0)
        for k in range(NBUF):
            wait_out(k)

    return sc_add(x, pe)
